# baseline (speedup 1.0000x reference)
import functools

import jax
import jax.numpy as jnp
from jax import lax
from jax.experimental import pallas as pl
from jax.experimental.pallas import tpu as pltpu
from jax.experimental.pallas import tpu_sc as plsc

V = 1000000
D = 32
NW = 32
R4 = 250000
NFULL = 7812
TAIL0 = NFULL * 128
K1_STEPS = 123

_params = pltpu.CompilerParams(
    use_tc_tiling_on_sc=True, needs_layout_passes=False)
_mesh = plsc.VectorSubcoreMesh(core_axis_name="c", subcore_axis_name="s")


def _repack_table(table_t, tail4):
    @functools.partial(
        pl.kernel,
        mesh=_mesh,
        out_type=jax.ShapeDtypeStruct((R4, 128), jnp.float32),
        compiler_params=_params,
        scratch_types=[
            pltpu.VMEM((2, 32, 128), jnp.float32),
            pltpu.VMEM((2, 32, 128), jnp.float32),
            pltpu.VMEM((16, 128), jnp.float32),
            [pltpu.SemaphoreType.DMA] * 2,
            [pltpu.SemaphoreType.DMA] * 2,
            pltpu.SemaphoreType.DMA,
        ],
    )
    def k1(tt_hbm, tail_hbm, out_hbm, in_v, out_v, tail_v, isems, osems, tsem):
        wid = lax.axis_index("s") * 2 + lax.axis_index("c")
        iota16 = lax.iota(jnp.int32, 16)
        rows_lo = iota16
        rows_hi = iota16 + 16

        def in_copy(t, b):
            blk = wid + 32 * t
            v0 = pl.multiple_of(blk * 128, 128)
            return pltpu.make_async_copy(
                tt_hbm.at[:, pl.ds(v0, 128)], in_v.at[b], isems[b])

        def out_copy(t, b):
            blk = wid + 32 * t
            r0 = pl.multiple_of(blk * 32, 32)
            return pltpu.make_async_copy(
                out_v.at[b], out_hbm.at[pl.ds(r0, 32)], osems[b])

        def valid(t):
            return (wid + 32 * t) < NFULL

        for b in range(2):
            @pl.when(valid(b))
            def _():
                in_copy(b, b).start()

        def body(s, carry):
            for b in range(2):
                t = 2 * s + b

                @pl.when(valid(t))
                def _():
                    in_copy(t, b).wait()

                    @pl.when(s > 0)
                    def _():
                        out_copy(t - 2, b).wait()

                    for r in range(32):
                        for l in range(8):
                            j0 = 4 * r + l // 2
                            rows = rows_lo if l % 2 == 0 else rows_hi
                            cols = jnp.full((16,), j0, jnp.int32)
                            out_v[b, r, pl.ds(16 * l, 16)] = plsc.load_gather(
                                in_v.at[b], [rows, cols])
                    out_copy(t, b).start()

                    @pl.when(valid(t + 2))
                    def _():
                        in_copy(t + 2, b).start()

            return carry

        lax.fori_loop(0, K1_STEPS, body, 0)

        for b in range(2):
            t_last = 2 * (K1_STEPS - 1) + b

            @pl.when(valid(t_last))
            def _():
                out_copy(t_last, b).wait()

            @pl.when(jnp.logical_and(valid(t_last - 2),
                                     jnp.logical_not(valid(t_last))))
            def _():
                out_copy(t_last - 2, b).wait()

        @pl.when(wid == 31)
        def _():
            pltpu.sync_copy(tail_hbm, tail_v)
            pltpu.sync_copy(tail_v, out_hbm.at[pl.ds(NFULL * 32, 16)])

    return k1(table_t, tail4)


def _gather(table4, flat_idx):
    NCH = 104

    @functools.partial(
        pl.kernel,
        mesh=_mesh,
        out_type=jax.ShapeDtypeStruct((26, 32, 16384), jnp.float32),
        compiler_params=_params,
        scratch_types=[
            pltpu.VMEM((13312,), jnp.int32),
            pltpu.VMEM((2, 128), jnp.int32),
            pltpu.VMEM((2, 128), jnp.int32),
            pltpu.VMEM((2, 128, 128), jnp.float32),
            pltpu.VMEM((2, 32, 128), jnp.float32),
            pltpu.SemaphoreType.DMA,
            [pltpu.SemaphoreType.DMA] * 2,
            [pltpu.SemaphoreType.DMA] * 2,
        ],
    )
    def k2(t4_hbm, idx_hbm, out_hbm, idx_v, row4_v, sub32_v, wide_v, outb_v,
           isem, gsems, wsems):
        wid = lax.axis_index("s") * 2 + lax.axis_index("c")
        iota16 = lax.iota(jnp.int32, 16)
        iota26 = iota16 * 26
        base_i = pl.multiple_of(wid * 13312, 1024)
        pltpu.make_async_copy(
            idx_hbm.at[pl.ds(base_i, 13312)], idx_v, isem).start()
        pltpu.make_async_copy(
            idx_hbm.at[pl.ds(base_i, 13312)], idx_v, isem).wait()

        def prep(c, b):
            f = c // 4
            q = c % 4
            for t in range(8):
                pos = iota26 + ((128 * q + 16 * t) * 26 + f)
                idx16 = plsc.load_gather(idx_v, [pos])
                row4_v[b, pl.ds(16 * t, 16)] = idx16 >> 2
                sub32_v[b, pl.ds(16 * t, 16)] = (idx16 & 3) << 5

        def g_copy(b):
            return pltpu.make_async_copy(
                t4_hbm.at[row4_v.at[b]], wide_v.at[b], gsems[b])

        def w_copy(c, b):
            f = c // 4
            q = c % 4
            o = pl.multiple_of(wid * 512 + q * 128, 128)
            return pltpu.make_async_copy(
                outb_v.at[b], out_hbm.at[f, :, pl.ds(o, 128)], wsems[b])

        for b in range(2):
            prep(b, b)
            g_copy(b).start()

        def body(s, carry):
            for b in range(2):
                c = 2 * s + b
                g_copy(b).wait()

                @pl.when(s > 0)
                def _():
                    w_copy(c - 2, b).wait()

                for t in range(8):
                    rows = iota16 + 16 * t
                    sub32 = sub32_v[b, pl.ds(16 * t, 16)]
                    for d in range(32):
                        outb_v[b, d, pl.ds(16 * t, 16)] = plsc.load_gather(
                            wide_v.at[b], [rows, sub32 + d])
                w_copy(c, b).start()

                @pl.when(c + 2 < NCH)
                def _():
                    prep(c + 2, b)
                    g_copy(b).start()

            return carry

        lax.fori_loop(0, NCH // 2, body, 0)
        for b in range(2):
            w_copy(NCH - 2 + b, b).wait()

    return k2(table4, flat_idx)


def kernel(word_index, embed_weight):
    bsz, f = word_index.shape
    flat = jnp.reshape(word_index.astype(jnp.int32), (bsz * f,))
    table_t = jnp.transpose(embed_weight)
    tail4 = jnp.reshape(embed_weight[TAIL0:], (16, 128))
    table4 = _repack_table(table_t, tail4)
    out3 = _gather(table4, flat)
    return jnp.transpose(out3, (2, 0, 1))


# speedup vs baseline: 1.7599x; 1.7599x over previous
import functools

import jax
import jax.numpy as jnp
from jax import lax
from jax.experimental import pallas as pl
from jax.experimental.pallas import tpu as pltpu
from jax.experimental.pallas import tpu_sc as plsc

NFULL = 7812
TAIL0 = NFULL * 128
K1_NB = 4
K1_STEPS = 62
K2_NB = 4
NCH = 104

_params = pltpu.CompilerParams(
    use_tc_tiling_on_sc=True, needs_layout_passes=False)
_mesh = plsc.VectorSubcoreMesh(core_axis_name="c", subcore_axis_name="s")


def _repack_table(table_t, tail4):
    @functools.partial(
        pl.kernel,
        mesh=_mesh,
        out_type=jax.ShapeDtypeStruct((250000, 128), jnp.float32),
        compiler_params=_params,
        scratch_types=[
            pltpu.VMEM((K1_NB, 32, 128), jnp.float32),
            pltpu.VMEM((K1_NB, 32, 128), jnp.float32),
            pltpu.VMEM((16, 128), jnp.float32),
            [pltpu.SemaphoreType.DMA] * K1_NB,
            [pltpu.SemaphoreType.DMA] * K1_NB,
        ],
    )
    def k1(tt_hbm, tail_hbm, out_hbm, in_v, out_v, tail_v, isems, osems):
        wid = lax.axis_index("s") * 2 + lax.axis_index("c")
        iota16 = lax.iota(jnp.int32, 16)
        rowp = []
        colp = []
        for jb in range(8):
            jj = iota16 + 16 * jb
            rowp.append(jj >> 2)
            colp.append((jj & 3) << 5)

        def in_copy(t, b):
            v0 = pl.multiple_of((wid + 32 * t) * 128, 128)
            return pltpu.make_async_copy(
                tt_hbm.at[:, pl.ds(v0, 128)], in_v.at[b], isems[b])

        def out_copy(t, b):
            r0 = pl.multiple_of((wid + 32 * t) * 32, 32)
            return pltpu.make_async_copy(
                out_v.at[b], out_hbm.at[pl.ds(r0, 32)], osems[b])

        def valid(t):
            return (wid + 32 * t) < NFULL

        for b in range(K1_NB):
            @pl.when(valid(b))
            def _():
                in_copy(b, b).start()

        def body(s, carry):
            for b in range(K1_NB):
                t = K1_NB * s + b

                @pl.when(valid(t))
                def _():
                    in_copy(t, b).wait()

                    @pl.when(s > 0)
                    def _():
                        out_copy(t - K1_NB, b).wait()

                    @plsc.parallel_loop(0, 32, unroll=2)
                    def _(d):
                        for jb in range(8):
                            x = in_v[b, d, pl.ds(16 * jb, 16)]
                            plsc.store_scatter(
                                out_v.at[b], [rowp[jb], colp[jb] + d], x)

                    out_copy(t, b).start()

                    @pl.when(valid(t + K1_NB))
                    def _():
                        in_copy(t + K1_NB, b).start()

            return carry

        lax.fori_loop(0, K1_STEPS, body, 0)

        for b in range(K1_NB):
            t1 = (K1_STEPS - 1) * K1_NB + b

            @pl.when(valid(t1))
            def _():
                out_copy(t1, b).wait()

            @pl.when(jnp.logical_and(jnp.logical_not(valid(t1)),
                                     valid(t1 - K1_NB)))
            def _():
                out_copy(t1 - K1_NB, b).wait()

        @pl.when(wid == 31)
        def _():
            pltpu.sync_copy(tail_hbm, tail_v)
            pltpu.sync_copy(tail_v, out_hbm.at[pl.ds(NFULL * 32, 16)])

    return k1(table_t, tail4)


def _gather(table4, flat_idx):
    @functools.partial(
        pl.kernel,
        mesh=_mesh,
        out_type=jax.ShapeDtypeStruct((26, 32, 16384), jnp.float32),
        compiler_params=_params,
        scratch_types=[
            pltpu.VMEM((13312,), jnp.int32),
            pltpu.VMEM((K2_NB, 128), jnp.int32),
            pltpu.VMEM((K2_NB, 128), jnp.int32),
            pltpu.VMEM((K2_NB, 128, 128), jnp.float32),
            pltpu.VMEM((K2_NB, 32, 128), jnp.float32),
            pltpu.SemaphoreType.DMA,
            [pltpu.SemaphoreType.DMA] * K2_NB,
            [pltpu.SemaphoreType.DMA] * K2_NB,
        ],
    )
    def k2(t4_hbm, idx_hbm, out_hbm, idx_v, row4_v, sub32_v, wide_v, outb_v,
           isem, gsems, wsems):
        wid = lax.axis_index("s") * 2 + lax.axis_index("c")
        iota16 = lax.iota(jnp.int32, 16)
        iota26 = iota16 * 26
        base_i = pl.multiple_of(wid * 13312, 1024)
        pltpu.make_async_copy(
            idx_hbm.at[pl.ds(base_i, 13312)], idx_v, isem).start()
        pltpu.make_async_copy(
            idx_hbm.at[pl.ds(base_i, 13312)], idx_v, isem).wait()

        def prep(c, b):
            f = c // 4
            q = c % 4
            for t in range(8):
                pos = iota26 + ((128 * q + 16 * t) * 26 + f)
                idx16 = plsc.load_gather(idx_v, [pos])
                row4_v[b, pl.ds(16 * t, 16)] = idx16 >> 2
                sub32_v[b, pl.ds(16 * t, 16)] = (idx16 & 3) << 5

        def g_copy(b):
            return pltpu.make_async_copy(
                t4_hbm.at[row4_v.at[b]], wide_v.at[b], gsems[b])

        def w_copy(c, b):
            f = c // 4
            q = c % 4
            o = pl.multiple_of(wid * 512 + q * 128, 128)
            return pltpu.make_async_copy(
                outb_v.at[b], out_hbm.at[f, :, pl.ds(o, 128)], wsems[b])

        for b in range(K2_NB):
            prep(b, b)
            g_copy(b).start()

        def body(s, carry):
            for b in range(K2_NB):
                c = K2_NB * s + b
                g_copy(b).wait()

                @pl.when(s > 0)
                def _():
                    w_copy(c - K2_NB, b).wait()

                for t in range(8):
                    rows = iota16 + 16 * t
                    sub32 = sub32_v[b, pl.ds(16 * t, 16)]

                    @plsc.parallel_loop(0, 32, unroll=4)
                    def _(d):
                        outb_v[b, d, pl.ds(16 * t, 16)] = plsc.load_gather(
                            wide_v.at[b], [rows, sub32 + d])

                w_copy(c, b).start()

                @pl.when(c + K2_NB < NCH)
                def _():
                    prep(c + K2_NB, b)
                    g_copy(b).start()

            return carry

        lax.fori_loop(0, NCH // K2_NB, body, 0)
        for b in range(K2_NB):
            w_copy(NCH - K2_NB + b, b).wait()

    return k2(table4, flat_idx)


def kernel(word_index, embed_weight):
    bsz, nf = word_index.shape
    flat = jnp.reshape(word_index.astype(jnp.int32), (bsz * nf,))
    table_t = jnp.transpose(embed_weight)
    tail4 = jnp.reshape(embed_weight[TAIL0:], (16, 128))
    table4 = _repack_table(table_t, tail4)
    out3 = _gather(table4, flat)
    return jnp.transpose(out3, (2, 0, 1))
